# baseline (speedup 1.0000x reference)
import functools

import jax
import jax.numpy as jnp
from jax import lax
from jax.experimental import pallas as pl
from jax.experimental.pallas import tpu as pltpu
from jax.experimental.pallas import tpu_sc as plsc

_B = 16
_P = 1024
_D = 1024
_ROWS = _B * _P

_NW = 32
_SC_ROWS = 4096
_RPW = _SC_ROWS // _NW
_FPW = _RPW // 16

_TC_ROWS = _ROWS - _SC_ROWS
_TC_BLOCK = 1024

_mesh = plsc.VectorSubcoreMesh(core_axis_name="c", subcore_axis_name="s")


@functools.partial(
    pl.kernel,
    mesh=_mesh,
    out_type=jax.ShapeDtypeStruct((_SC_ROWS, _D), jnp.float32),
    scratch_types=[
        pltpu.VMEM((_FPW,), jnp.int32),
        pltpu.VMEM((_FPW, _D), jnp.float32),
        pltpu.VMEM_SHARED((_RPW, _D), jnp.float32),
        pltpu.SemaphoreType.DMA,
        pltpu.SemaphoreType.DMA,
    ],
)
def _sc_broadcast(idx_hbm, table_hbm, out_hbm, idx_v, buf_v, stage_s, gsem, osem):
    cid = lax.axis_index("c")
    sid = lax.axis_index("s")
    wid = sid * 2 + cid
    pltpu.sync_copy(idx_hbm, idx_v)
    pltpu.async_copy(table_hbm.at[idx_v], buf_v, gsem).wait()
    pltpu.sync_copy(buf_v, stage_s.at[pl.ds(sid * _FPW, _FPW)])
    plsc.subcore_barrier()
    pltpu.async_copy(stage_s, out_hbm.at[pl.ds(wid * _RPW, _RPW)], osem).wait()


def _tc_body(idx_ref, row_ref, out_ref):
    del idx_ref
    out_ref[...] = jnp.broadcast_to(row_ref[0], out_ref.shape)


def _tc_broadcast(idx, table):
    grid_spec = pltpu.PrefetchScalarGridSpec(
        num_scalar_prefetch=1,
        grid=(_TC_ROWS // _TC_BLOCK,),
        in_specs=[
            pl.BlockSpec((1, 1, _D), lambda i, idx_ref: (idx_ref[0], 0, 0)),
        ],
        out_specs=pl.BlockSpec((_TC_BLOCK, _D), lambda i, idx_ref: (i, 0)),
    )
    return pl.pallas_call(
        _tc_body,
        grid_spec=grid_spec,
        out_shape=jax.ShapeDtypeStruct((_TC_ROWS, _D), jnp.float32),
    )(idx.reshape(1), table.reshape(-1, 1, _D))


def kernel(scale_embed, batch_size, num_patches, scale_idx):
    dep = (jnp.asarray(batch_size) - _B) + (jnp.asarray(num_patches) - _P)
    idx = (jnp.asarray(scale_idx) + dep).astype(jnp.int32)
    sc2d = _sc_broadcast(jnp.broadcast_to(idx, (_FPW,)), scale_embed)
    tc2d = _tc_broadcast(idx, scale_embed)
    out2d = jnp.concatenate([tc2d, sc2d], axis=0)
    return out2d.reshape(_B, _P, _D)
